# baseline (speedup 1.0000x reference)
import functools

import jax
import jax.numpy as jnp
from jax.experimental import pallas as pl
from jax.experimental.pallas import tpu as pltpu
from jax.experimental.pallas import tpu_sc as plsc

_WINDOW = 56
_NUM_WORKERS = 32
_BN = 1000
_NCHUNK = 5


def _gather_children(h, c, idx_even, idx_odd, n_pad):
    n, hs = h.shape
    mesh = plsc.VectorSubcoreMesh(core_axis_name="core", subcore_axis_name="subcore")
    out_t = jax.ShapeDtypeStruct((n_pad, hs), h.dtype)

    @functools.partial(pl.kernel, out_type=(out_t, out_t, out_t, out_t), mesh=mesh,
                       scratch_types=[pltpu.SemaphoreType.DMA])
    def gather_kernel(h_hbm, c_hbm, ie_hbm, io_hbm, oh1, oh2, oc1, oc2, sem):
        def body(ie_v, io_v, oh1_v, oh2_v, oc1_v, oc2_v):
            a = pltpu.async_copy(h_hbm.at[ie_v.at[0, 0]], oh1_v, sem)
            b = pltpu.async_copy(h_hbm.at[io_v.at[0, 0]], oh2_v, sem)
            d = pltpu.async_copy(c_hbm.at[ie_v.at[0, 0]], oc1_v, sem)
            e = pltpu.async_copy(c_hbm.at[io_v.at[0, 0]], oc2_v, sem)
            a.wait()
            b.wait()
            d.wait()
            e.wait()

        pltpu.emit_pipeline(
            body,
            grid=(n_pad // _WINDOW,),
            in_specs=[
                pl.BlockSpec((1, 1, _WINDOW), lambda i: (i, 0, 0)),
                pl.BlockSpec((1, 1, _WINDOW), lambda i: (i, 0, 0)),
            ],
            out_specs=[
                pl.BlockSpec((_WINDOW, hs), lambda i: (i, 0)),
                pl.BlockSpec((_WINDOW, hs), lambda i: (i, 0)),
                pl.BlockSpec((_WINDOW, hs), lambda i: (i, 0)),
                pl.BlockSpec((_WINDOW, hs), lambda i: (i, 0)),
            ],
            core_axis_name=("core", "subcore"),
            dimension_semantics=(pltpu.PARALLEL,),
        )(ie_hbm, io_hbm, oh1, oh2, oc1, oc2)

    return gather_kernel(h, c, idx_even, idx_odd)


def _tc_body(scale_ref, iou_ref, h1_ref, h2_ref, c1_ref, c2_ref,
             wf1_ref, wf2_ref, wi1_ref, wi2_ref, bi_ref, fb_ref, *rest):
    out_ref = rest[-1]
    hs = h1_ref.shape[1]
    prec = jax.lax.Precision.DEFAULT
    h1 = h1_ref[...]
    h2 = h2_ref[...]
    c1 = c1_ref[...]
    c2 = c2_ref[...]

    fpre = (jnp.dot(h1, wf1_ref[...], preferred_element_type=jnp.float32, precision=prec)
            + jnp.dot(h2, wf2_ref[...], preferred_element_type=jnp.float32, precision=prec)
            + fb_ref[...])
    f = jax.nn.sigmoid(fpre)
    c_red = f[:, :hs] * c1 + f[:, hs:] * c2

    mm = (jnp.dot(h1, wi1_ref[...], preferred_element_type=jnp.float32, precision=prec)
          + jnp.dot(h2, wi2_ref[...], preferred_element_type=jnp.float32, precision=prec))

    hnorm = jnp.sqrt(jnp.sum(h1 * h1, axis=1, keepdims=True)
                     + jnp.sum(h2 * h2, axis=1, keepdims=True))
    iou_x = iou_ref[...]
    inorm = jnp.sqrt(jnp.sum(iou_x * iou_x, axis=1, keepdims=True))
    s_iou = scale_ref[0, 0]
    s_c = scale_ref[0, 1]
    alpha = inorm * s_iou / jnp.maximum(hnorm, 1e-12)
    iou_new = mm * alpha + bi_ref[...]

    crn = jnp.sqrt(jnp.sum(c_red * c_red, axis=1, keepdims=True))
    c1n = jnp.sqrt(jnp.sum(c1 * c1, axis=1, keepdims=True))
    c_agg = c_red / jnp.maximum(crn, 1e-12) * c1n * s_c

    ig = jax.nn.sigmoid(iou_new[:, :hs])
    og = jax.nn.sigmoid(iou_new[:, hs:2 * hs])
    ug = jnp.tanh(iou_new[:, 2 * hs:])
    c_out = ig * ug + c_agg
    h_out = og * jnp.tanh(c_out)
    out_ref[:, :hs] = h_out
    out_ref[:, hs:] = c_out


def _tc_compute(scales, iou, h1, h2, c1, c2, wf1, wf2, wi1, wi2, bi, fb,
                rows=None, block_offset=0, prev=None, out_rows=None,
                interpret=False):
    hs = h1.shape[1]
    bn = _BN
    n = rows
    out_rows = n if out_rows is None else out_rows
    grid = (n // bn,)
    in_specs = [
        pl.BlockSpec(memory_space=pltpu.MemorySpace.SMEM),
        pl.BlockSpec((bn, 3 * hs),
                     lambda i, off=block_offset: (i + off, 0)),
        pl.BlockSpec((bn, hs), lambda i: (i, 0)),
        pl.BlockSpec((bn, hs), lambda i: (i, 0)),
        pl.BlockSpec((bn, hs), lambda i: (i, 0)),
        pl.BlockSpec((bn, hs), lambda i: (i, 0)),
        pl.BlockSpec((hs, 2 * hs), lambda i: (0, 0)),
        pl.BlockSpec((hs, 2 * hs), lambda i: (0, 0)),
        pl.BlockSpec((hs, 3 * hs), lambda i: (0, 0)),
        pl.BlockSpec((hs, 3 * hs), lambda i: (0, 0)),
        pl.BlockSpec((1, 3 * hs), lambda i: (0, 0)),
        pl.BlockSpec((1, 2 * hs), lambda i: (0, 0)),
    ]
    args = [scales, iou, h1, h2, c1, c2, wf1, wf2, wi1, wi2, bi, fb]
    io_aliases = {}
    if prev is not None:
        in_specs.append(pl.BlockSpec(memory_space=pl.ANY))
        args.append(prev)
        io_aliases = {12: 0}
    return pl.pallas_call(
        _tc_body,
        grid=grid,
        in_specs=in_specs,
        out_specs=pl.BlockSpec((bn, 2 * hs),
                               lambda i, off=block_offset: (i + off, 0)),
        out_shape=jax.ShapeDtypeStruct((out_rows, 2 * hs), jnp.float32),
        input_output_aliases=io_aliases,
        interpret=interpret,
    )(*args)


def kernel(iou, h, c, edge_index, U_iou, b_iou, U_f_w, U_f_b, scale_iou, scale_c):
    n, hs = h.shape
    src = edge_index[0]
    idx_even = src[0::2]
    idx_odd = src[1::2]

    wf = U_f_w.T
    wi = U_iou.T
    wf1, wf2 = wf[:hs], wf[hs:]
    wi1, wi2 = wi[:hs], wi[hs:]
    fb = U_f_b.reshape(1, 2 * hs)
    scales = jnp.stack([scale_iou.astype(jnp.float32),
                        scale_c.astype(jnp.float32)]).reshape(1, 2)
    nchunk = _NCHUNK if n % (_NCHUNK * _BN) == 0 else 1
    cn = n // nchunk
    unit = _WINDOW * _NUM_WORKERS
    cn_pad = ((cn + unit - 1) // unit) * unit
    pad = cn_pad - cn
    steps = cn_pad // _WINDOW

    out = None
    for k in range(nchunk):
        se = jax.lax.slice(idx_even, (k * cn,), ((k + 1) * cn,))
        so = jax.lax.slice(idx_odd, (k * cn,), ((k + 1) * cn,))
        se = jnp.pad(se, (0, pad)).reshape(steps, 1, _WINDOW)
        so = jnp.pad(so, (0, pad)).reshape(steps, 1, _WINDOW)
        h1, h2, c1, c2 = _gather_children(h, c, se, so, cn_pad)
        out = _tc_compute(scales, iou, h1, h2, c1, c2,
                          wf1, wf2, wi1, wi2, b_iou, fb,
                          rows=cn, block_offset=k * (cn // _BN),
                          prev=out, out_rows=n)
    return out


# speedup vs baseline: 1.1941x; 1.1941x over previous
import functools

import jax
import jax.numpy as jnp
from jax.experimental import pallas as pl
from jax.experimental.pallas import tpu as pltpu
from jax.experimental.pallas import tpu_sc as plsc

_WINDOW = 56
_NUM_WORKERS = 32
_BN = 1000
_NCHUNK = 5


def _gather_children(h, c, idx_even, idx_odd, n_pad):
    n, hs = h.shape
    mesh = plsc.VectorSubcoreMesh(core_axis_name="core", subcore_axis_name="subcore")
    out_t = jax.ShapeDtypeStruct((n_pad, hs), h.dtype)

    @functools.partial(pl.kernel, out_type=(out_t, out_t, out_t, out_t), mesh=mesh,
                       scratch_types=[pltpu.SemaphoreType.DMA])
    def gather_kernel(h_hbm, c_hbm, ie_hbm, io_hbm, oh1, oh2, oc1, oc2, sem):
        def body(ie_v, io_v, oh1_v, oh2_v, oc1_v, oc2_v):
            a = pltpu.async_copy(h_hbm.at[ie_v.at[0, 0]], oh1_v, sem)
            b = pltpu.async_copy(h_hbm.at[io_v.at[0, 0]], oh2_v, sem)
            d = pltpu.async_copy(c_hbm.at[ie_v.at[0, 0]], oc1_v, sem)
            e = pltpu.async_copy(c_hbm.at[io_v.at[0, 0]], oc2_v, sem)
            a.wait()
            b.wait()
            d.wait()
            e.wait()

        pltpu.emit_pipeline(
            body,
            grid=(n_pad // _WINDOW,),
            in_specs=[
                pl.BlockSpec((1, 1, _WINDOW), lambda i: (i, 0, 0)),
                pl.BlockSpec((1, 1, _WINDOW), lambda i: (i, 0, 0)),
            ],
            out_specs=[
                pl.BlockSpec((_WINDOW, hs), lambda i: (i, 0)),
                pl.BlockSpec((_WINDOW, hs), lambda i: (i, 0)),
                pl.BlockSpec((_WINDOW, hs), lambda i: (i, 0)),
                pl.BlockSpec((_WINDOW, hs), lambda i: (i, 0)),
            ],
            core_axis_name=("core", "subcore"),
            dimension_semantics=(pltpu.PARALLEL,),
        )(ie_hbm, io_hbm, oh1, oh2, oc1, oc2)

    return gather_kernel(h, c, idx_even, idx_odd)


def _tc_body(scale_ref, iou_ref, h1_ref, h2_ref, c1_ref, c2_ref,
             wf1_ref, wf2_ref, wi1_ref, wi2_ref, bi_ref, fb_ref, *rest):
    out_ref = rest[-1]
    hs = h1_ref.shape[1]
    prec = jax.lax.Precision.DEFAULT
    h1 = h1_ref[...]
    h2 = h2_ref[...]
    c1 = c1_ref[...]
    c2 = c2_ref[...]

    fpre = (jnp.dot(h1, wf1_ref[...], preferred_element_type=jnp.float32, precision=prec)
            + jnp.dot(h2, wf2_ref[...], preferred_element_type=jnp.float32, precision=prec)
            + fb_ref[...])
    f = jax.nn.sigmoid(fpre)
    c_red = f[:, :hs] * c1 + f[:, hs:] * c2

    mm = (jnp.dot(h1, wi1_ref[...], preferred_element_type=jnp.float32, precision=prec)
          + jnp.dot(h2, wi2_ref[...], preferred_element_type=jnp.float32, precision=prec))

    hnorm = jnp.sqrt(jnp.sum(h1 * h1, axis=1, keepdims=True)
                     + jnp.sum(h2 * h2, axis=1, keepdims=True))
    iou_x = iou_ref[...]
    inorm = jnp.sqrt(jnp.sum(iou_x * iou_x, axis=1, keepdims=True))
    s_iou = scale_ref[0, 0]
    s_c = scale_ref[0, 1]
    alpha = inorm * s_iou / jnp.maximum(hnorm, 1e-12)
    iou_new = mm * alpha + bi_ref[...]

    crn = jnp.sqrt(jnp.sum(c_red * c_red, axis=1, keepdims=True))
    c1n = jnp.sqrt(jnp.sum(c1 * c1, axis=1, keepdims=True))
    c_agg = c_red / jnp.maximum(crn, 1e-12) * c1n * s_c

    ig = jax.nn.sigmoid(iou_new[:, :hs])
    og = jax.nn.sigmoid(iou_new[:, hs:2 * hs])
    ug = jnp.tanh(iou_new[:, 2 * hs:])
    c_out = ig * ug + c_agg
    h_out = og * jnp.tanh(c_out)
    out_ref[:, :hs] = h_out
    out_ref[:, hs:] = c_out


def _tc_compute(scales, iou, h1, h2, c1, c2, wf1, wf2, wi1, wi2, bi, fb,
                rows=None, block_offset=0, prev=None, out_rows=None,
                interpret=False):
    hs = h1.shape[1]
    bn = _BN
    n = rows
    out_rows = n if out_rows is None else out_rows
    grid = (n // bn,)
    in_specs = [
        pl.BlockSpec(memory_space=pltpu.MemorySpace.SMEM),
        pl.BlockSpec((bn, 3 * hs),
                     lambda i, off=block_offset: (i + off, 0)),
        pl.BlockSpec((bn, hs), lambda i: (i, 0)),
        pl.BlockSpec((bn, hs), lambda i: (i, 0)),
        pl.BlockSpec((bn, hs), lambda i: (i, 0)),
        pl.BlockSpec((bn, hs), lambda i: (i, 0)),
        pl.BlockSpec((hs, 2 * hs), lambda i: (0, 0)),
        pl.BlockSpec((hs, 2 * hs), lambda i: (0, 0)),
        pl.BlockSpec((hs, 3 * hs), lambda i: (0, 0)),
        pl.BlockSpec((hs, 3 * hs), lambda i: (0, 0)),
        pl.BlockSpec((1, 3 * hs), lambda i: (0, 0)),
        pl.BlockSpec((1, 2 * hs), lambda i: (0, 0)),
    ]
    args = [scales, iou, h1, h2, c1, c2, wf1, wf2, wi1, wi2, bi, fb]
    io_aliases = {}
    if prev is not None:
        in_specs.append(pl.BlockSpec(memory_space=pl.ANY))
        args.append(prev)
        io_aliases = {12: 0}
    return pl.pallas_call(
        _tc_body,
        grid=grid,
        in_specs=in_specs,
        out_specs=pl.BlockSpec((bn, 2 * hs),
                               lambda i, off=block_offset: (i + off, 0)),
        out_shape=jax.ShapeDtypeStruct((out_rows, 2 * hs), jnp.float32),
        input_output_aliases=io_aliases,
        interpret=interpret,
    )(*args)


def kernel(iou, h, c, edge_index, U_iou, b_iou, U_f_w, U_f_b, scale_iou, scale_c):
    n, hs = h.shape
    src = edge_index[0]
    idx_even = src[0::2]
    idx_odd = src[1::2]

    wf = U_f_w.T
    wi = U_iou.T
    wf1, wf2 = wf[:hs], wf[hs:]
    wi1, wi2 = wi[:hs], wi[hs:]
    fb = U_f_b.reshape(1, 2 * hs)
    scales = jnp.stack([scale_iou.astype(jnp.float32),
                        scale_c.astype(jnp.float32)]).reshape(1, 2)
    if n % (10 * _BN) == 0:
        tenth = n // 10
        sizes = [tenth, 4 * tenth, 4 * tenth, tenth]
    else:
        sizes = [n]
    unit = _WINDOW * _NUM_WORKERS

    out = None
    base = 0
    for cn in sizes:
        cn_pad = ((cn + unit - 1) // unit) * unit
        pad = cn_pad - cn
        steps = cn_pad // _WINDOW
        se = jax.lax.slice(idx_even, (base,), (base + cn,))
        so = jax.lax.slice(idx_odd, (base,), (base + cn,))
        se = jnp.pad(se, (0, pad)).reshape(steps, 1, _WINDOW)
        so = jnp.pad(so, (0, pad)).reshape(steps, 1, _WINDOW)
        h1, h2, c1, c2 = _gather_children(h, c, se, so, cn_pad)
        out = _tc_compute(scales, iou, h1, h2, c1, c2,
                          wf1, wf2, wi1, wi2, b_iou, fb,
                          rows=cn, block_offset=base // _BN,
                          prev=out, out_rows=n)
        base += cn
    return out


# speedup vs baseline: 2.3563x; 1.9733x over previous
import functools

import jax
import jax.numpy as jnp
from jax import lax
from jax.experimental import pallas as pl
from jax.experimental.pallas import tpu as pltpu
from jax.experimental.pallas import tpu_sc as plsc

_WINDOW = 112
_NUM_WORKERS = 32
_BN = 1000
_NCHUNK = 4
_HI = 0xFFFF0000
_RND = 0x8000


def _pack_body(h_ref, c_ref, o_ref):
    uh = lax.bitcast_convert_type(h_ref[...], jnp.uint32)
    uc = lax.bitcast_convert_type(c_ref[...], jnp.uint32)
    lo = (uh + jnp.uint32(_RND)) >> 16
    hi = (uc + jnp.uint32(_RND)) & jnp.uint32(_HI)
    o_ref[...] = lax.bitcast_convert_type(lo | hi, jnp.float32)


def _pack_hc(h, c, bn=2000):
    n, hs = h.shape
    return pl.pallas_call(
        _pack_body,
        grid=(n // bn,),
        in_specs=[pl.BlockSpec((bn, hs), lambda i: (i, 0)),
                  pl.BlockSpec((bn, hs), lambda i: (i, 0))],
        out_specs=pl.BlockSpec((bn, hs), lambda i: (i, 0)),
        out_shape=jax.ShapeDtypeStruct((n, hs), jnp.float32),
    )(h, c)


def _gather_children(p, idx_even, idx_odd, n_pad):
    n, hs = p.shape
    mesh = plsc.VectorSubcoreMesh(core_axis_name="core", subcore_axis_name="subcore")
    out_t = jax.ShapeDtypeStruct((n_pad, hs), p.dtype)

    @functools.partial(pl.kernel, out_type=(out_t, out_t), mesh=mesh,
                       scratch_types=[pltpu.SemaphoreType.DMA])
    def gather_kernel(p_hbm, ie_hbm, io_hbm, o1, o2, sem):
        def body(ie_v, io_v, o1_v, o2_v):
            a = pltpu.async_copy(p_hbm.at[ie_v.at[0, 0]], o1_v, sem)
            b = pltpu.async_copy(p_hbm.at[io_v.at[0, 0]], o2_v, sem)
            a.wait()
            b.wait()

        pltpu.emit_pipeline(
            body,
            grid=(n_pad // _WINDOW,),
            in_specs=[
                pl.BlockSpec((1, 1, _WINDOW), lambda i: (i, 0, 0)),
                pl.BlockSpec((1, 1, _WINDOW), lambda i: (i, 0, 0)),
            ],
            out_specs=[
                pl.BlockSpec((_WINDOW, hs), lambda i: (i, 0)),
                pl.BlockSpec((_WINDOW, hs), lambda i: (i, 0)),
            ],
            core_axis_name=("core", "subcore"),
            dimension_semantics=(pltpu.PARALLEL,),
        )(ie_hbm, io_hbm, o1, o2)

    return gather_kernel(p, idx_even, idx_odd)


def _tc_body(scale_ref, iou_ref, p1_ref, p2_ref,
             wf1_ref, wf2_ref, wi1_ref, wi2_ref, bi_ref, fb_ref, *rest):
    out_ref = rest[-1]
    hs = p1_ref.shape[1]
    prec = jax.lax.Precision.DEFAULT
    u1 = lax.bitcast_convert_type(p1_ref[...], jnp.uint32)
    u2 = lax.bitcast_convert_type(p2_ref[...], jnp.uint32)
    h1 = lax.bitcast_convert_type(u1 << 16, jnp.float32)
    c1 = lax.bitcast_convert_type(u1 & jnp.uint32(_HI), jnp.float32)
    h2 = lax.bitcast_convert_type(u2 << 16, jnp.float32)
    c2 = lax.bitcast_convert_type(u2 & jnp.uint32(_HI), jnp.float32)

    fpre = (jnp.dot(h1, wf1_ref[...], preferred_element_type=jnp.float32, precision=prec)
            + jnp.dot(h2, wf2_ref[...], preferred_element_type=jnp.float32, precision=prec)
            + fb_ref[...])
    f = jax.nn.sigmoid(fpre)
    c_red = f[:, :hs] * c1 + f[:, hs:] * c2

    mm = (jnp.dot(h1, wi1_ref[...], preferred_element_type=jnp.float32, precision=prec)
          + jnp.dot(h2, wi2_ref[...], preferred_element_type=jnp.float32, precision=prec))

    hnorm = jnp.sqrt(jnp.sum(h1 * h1, axis=1, keepdims=True)
                     + jnp.sum(h2 * h2, axis=1, keepdims=True))
    iou_x = iou_ref[...]
    inorm = jnp.sqrt(jnp.sum(iou_x * iou_x, axis=1, keepdims=True))
    s_iou = scale_ref[0, 0]
    s_c = scale_ref[0, 1]
    alpha = inorm * s_iou / jnp.maximum(hnorm, 1e-12)
    iou_new = mm * alpha + bi_ref[...]

    crn = jnp.sqrt(jnp.sum(c_red * c_red, axis=1, keepdims=True))
    c1n = jnp.sqrt(jnp.sum(c1 * c1, axis=1, keepdims=True))
    c_agg = c_red / jnp.maximum(crn, 1e-12) * c1n * s_c

    ig = jax.nn.sigmoid(iou_new[:, :hs])
    og = jax.nn.sigmoid(iou_new[:, hs:2 * hs])
    ug = jnp.tanh(iou_new[:, 2 * hs:])
    c_out = ig * ug + c_agg
    h_out = og * jnp.tanh(c_out)
    out_ref[:, :hs] = h_out
    out_ref[:, hs:] = c_out


def _tc_compute(scales, iou, p1, p2, wf1, wf2, wi1, wi2, bi, fb,
                rows=None, block_offset=0, prev=None, out_rows=None,
                interpret=False):
    hs = p1.shape[1]
    bn = _BN
    n = rows
    out_rows = n if out_rows is None else out_rows
    grid = (n // bn,)
    in_specs = [
        pl.BlockSpec(memory_space=pltpu.MemorySpace.SMEM),
        pl.BlockSpec((bn, 3 * hs),
                     lambda i, off=block_offset: (i + off, 0)),
        pl.BlockSpec((bn, hs), lambda i: (i, 0)),
        pl.BlockSpec((bn, hs), lambda i: (i, 0)),
        pl.BlockSpec((hs, 2 * hs), lambda i: (0, 0)),
        pl.BlockSpec((hs, 2 * hs), lambda i: (0, 0)),
        pl.BlockSpec((hs, 3 * hs), lambda i: (0, 0)),
        pl.BlockSpec((hs, 3 * hs), lambda i: (0, 0)),
        pl.BlockSpec((1, 3 * hs), lambda i: (0, 0)),
        pl.BlockSpec((1, 2 * hs), lambda i: (0, 0)),
    ]
    args = [scales, iou, p1, p2, wf1, wf2, wi1, wi2, bi, fb]
    io_aliases = {}
    if prev is not None:
        in_specs.append(pl.BlockSpec(memory_space=pl.ANY))
        args.append(prev)
        io_aliases = {10: 0}
    return pl.pallas_call(
        _tc_body,
        grid=grid,
        in_specs=in_specs,
        out_specs=pl.BlockSpec((bn, 2 * hs),
                               lambda i, off=block_offset: (i + off, 0)),
        out_shape=jax.ShapeDtypeStruct((out_rows, 2 * hs), jnp.float32),
        input_output_aliases=io_aliases,
        interpret=interpret,
    )(*args)


def kernel(iou, h, c, edge_index, U_iou, b_iou, U_f_w, U_f_b, scale_iou, scale_c):
    n, hs = h.shape
    src = edge_index[0]
    idx_even = src[0::2]
    idx_odd = src[1::2]

    wf = U_f_w.T
    wi = U_iou.T
    wf1, wf2 = wf[:hs], wf[hs:]
    wi1, wi2 = wi[:hs], wi[hs:]
    fb = U_f_b.reshape(1, 2 * hs)
    scales = jnp.stack([scale_iou.astype(jnp.float32),
                        scale_c.astype(jnp.float32)]).reshape(1, 2)

    p = _pack_hc(h, c)

    nchunk = _NCHUNK if n % (_NCHUNK * _BN) == 0 else 1
    cn = n // nchunk
    unit = _WINDOW * _NUM_WORKERS
    cn_pad = ((cn + unit - 1) // unit) * unit
    pad = cn_pad - cn
    steps = cn_pad // _WINDOW

    out = None
    for k in range(nchunk):
        se = jax.lax.slice(idx_even, (k * cn,), ((k + 1) * cn,))
        so = jax.lax.slice(idx_odd, (k * cn,), ((k + 1) * cn,))
        se = jnp.pad(se, (0, pad)).reshape(steps, 1, _WINDOW)
        so = jnp.pad(so, (0, pad)).reshape(steps, 1, _WINDOW)
        p1, p2 = _gather_children(p, se, so, cn_pad)
        out = _tc_compute(scales, iou, p1, p2,
                          wf1, wf2, wi1, wi2, b_iou, fb,
                          rows=cn, block_offset=k * (cn // _BN),
                          prev=out, out_rows=n)
    return out


# speedup vs baseline: 2.3803x; 1.0102x over previous
import functools

import jax
import jax.numpy as jnp
from jax import lax
from jax.experimental import pallas as pl
from jax.experimental.pallas import tpu as pltpu
from jax.experimental.pallas import tpu_sc as plsc

_WINDOW = 112
_NUM_WORKERS = 32
_BN = 1000
_NCHUNK = 4
_HI = 0xFFFF0000
_RND = 0x8000


def _pack_body(h_ref, c_ref, o_ref):
    uh = lax.bitcast_convert_type(h_ref[...], jnp.uint32)
    uc = lax.bitcast_convert_type(c_ref[...], jnp.uint32)
    lo = (uh + jnp.uint32(_RND)) >> 16
    hi = (uc + jnp.uint32(_RND)) & jnp.uint32(_HI)
    o_ref[...] = lax.bitcast_convert_type(lo | hi, jnp.float32)


def _pack_hc(h, c, bn=2000):
    n, hs = h.shape
    return pl.pallas_call(
        _pack_body,
        grid=(n // bn,),
        in_specs=[pl.BlockSpec((bn, hs), lambda i: (i, 0)),
                  pl.BlockSpec((bn, hs), lambda i: (i, 0))],
        out_specs=pl.BlockSpec((bn, hs), lambda i: (i, 0)),
        out_shape=jax.ShapeDtypeStruct((n, hs), jnp.float32),
    )(h, c)


def _gather_children(p, idx_even, idx_odd, n_pad):
    n, hs = p.shape
    mesh = plsc.VectorSubcoreMesh(core_axis_name="core", subcore_axis_name="subcore")
    out_t = jax.ShapeDtypeStruct((n_pad, hs), p.dtype)

    @functools.partial(pl.kernel, out_type=(out_t, out_t), mesh=mesh,
                       scratch_types=[pltpu.SemaphoreType.DMA])
    def gather_kernel(p_hbm, ie_hbm, io_hbm, o1, o2, sem):
        def body(ie_v, io_v, o1_v, o2_v):
            a = pltpu.async_copy(p_hbm.at[ie_v.at[0, 0]], o1_v, sem)
            b = pltpu.async_copy(p_hbm.at[io_v.at[0, 0]], o2_v, sem)
            a.wait()
            b.wait()

        pltpu.emit_pipeline(
            body,
            grid=(n_pad // _WINDOW,),
            in_specs=[
                pl.BlockSpec((1, 1, _WINDOW), lambda i: (i, 0, 0)),
                pl.BlockSpec((1, 1, _WINDOW), lambda i: (i, 0, 0)),
            ],
            out_specs=[
                pl.BlockSpec((_WINDOW, hs), lambda i: (i, 0)),
                pl.BlockSpec((_WINDOW, hs), lambda i: (i, 0)),
            ],
            core_axis_name=("core", "subcore"),
            dimension_semantics=(pltpu.PARALLEL,),
        )(ie_hbm, io_hbm, o1, o2)

    return gather_kernel(p, idx_even, idx_odd)


def _tc_body(scale_ref, iou_ref, p1_ref, p2_ref,
             wf1_ref, wf2_ref, wi1_ref, wi2_ref, bi_ref, fb_ref, *rest):
    out_ref = rest[-1]
    hs = p1_ref.shape[1]
    prec = jax.lax.Precision.DEFAULT
    u1 = lax.bitcast_convert_type(p1_ref[...], jnp.uint32)
    u2 = lax.bitcast_convert_type(p2_ref[...], jnp.uint32)
    h1 = lax.bitcast_convert_type(u1 << 16, jnp.float32)
    c1 = lax.bitcast_convert_type(u1 & jnp.uint32(_HI), jnp.float32)
    h2 = lax.bitcast_convert_type(u2 << 16, jnp.float32)
    c2 = lax.bitcast_convert_type(u2 & jnp.uint32(_HI), jnp.float32)

    h1b = h1.astype(jnp.bfloat16)
    h2b = h2.astype(jnp.bfloat16)
    fpre = (jnp.dot(h1b, wf1_ref[...], preferred_element_type=jnp.float32, precision=prec)
            + jnp.dot(h2b, wf2_ref[...], preferred_element_type=jnp.float32, precision=prec)
            + fb_ref[...])
    f = jax.nn.sigmoid(fpre)
    c_red = f[:, :hs] * c1 + f[:, hs:] * c2

    mm = (jnp.dot(h1b, wi1_ref[...], preferred_element_type=jnp.float32, precision=prec)
          + jnp.dot(h2b, wi2_ref[...], preferred_element_type=jnp.float32, precision=prec))

    hnorm = jnp.sqrt(jnp.sum(h1 * h1, axis=1, keepdims=True)
                     + jnp.sum(h2 * h2, axis=1, keepdims=True))
    iou_x = iou_ref[...]
    inorm = jnp.sqrt(jnp.sum(iou_x * iou_x, axis=1, keepdims=True))
    s_iou = scale_ref[0, 0]
    s_c = scale_ref[0, 1]
    alpha = inorm * s_iou / jnp.maximum(hnorm, 1e-12)
    iou_new = mm * alpha + bi_ref[...]

    crn = jnp.sqrt(jnp.sum(c_red * c_red, axis=1, keepdims=True))
    c1n = jnp.sqrt(jnp.sum(c1 * c1, axis=1, keepdims=True))
    c_agg = c_red / jnp.maximum(crn, 1e-12) * c1n * s_c

    ig = jax.nn.sigmoid(iou_new[:, :hs])
    og = jax.nn.sigmoid(iou_new[:, hs:2 * hs])
    ug = jnp.tanh(iou_new[:, 2 * hs:])
    c_out = ig * ug + c_agg
    h_out = og * jnp.tanh(c_out)
    out_ref[:, :hs] = h_out
    out_ref[:, hs:] = c_out


def _tc_compute(scales, iou, p1, p2, wf1, wf2, wi1, wi2, bi, fb,
                rows=None, block_offset=0, prev=None, out_rows=None,
                interpret=False):
    hs = p1.shape[1]
    bn = _BN
    n = rows
    out_rows = n if out_rows is None else out_rows
    grid = (n // bn,)
    in_specs = [
        pl.BlockSpec(memory_space=pltpu.MemorySpace.SMEM),
        pl.BlockSpec((bn, 3 * hs),
                     lambda i, off=block_offset: (i + off, 0)),
        pl.BlockSpec((bn, hs), lambda i: (i, 0)),
        pl.BlockSpec((bn, hs), lambda i: (i, 0)),
        pl.BlockSpec((hs, 2 * hs), lambda i: (0, 0)),
        pl.BlockSpec((hs, 2 * hs), lambda i: (0, 0)),
        pl.BlockSpec((hs, 3 * hs), lambda i: (0, 0)),
        pl.BlockSpec((hs, 3 * hs), lambda i: (0, 0)),
        pl.BlockSpec((1, 3 * hs), lambda i: (0, 0)),
        pl.BlockSpec((1, 2 * hs), lambda i: (0, 0)),
    ]
    args = [scales, iou, p1, p2, wf1, wf2, wi1, wi2, bi, fb]
    io_aliases = {}
    if prev is not None:
        in_specs.append(pl.BlockSpec(memory_space=pl.ANY))
        args.append(prev)
        io_aliases = {10: 0}
    return pl.pallas_call(
        _tc_body,
        grid=grid,
        in_specs=in_specs,
        out_specs=pl.BlockSpec((bn, 2 * hs),
                               lambda i, off=block_offset: (i + off, 0)),
        out_shape=jax.ShapeDtypeStruct((out_rows, 2 * hs), jnp.float32),
        input_output_aliases=io_aliases,
        interpret=interpret,
    )(*args)


def kernel(iou, h, c, edge_index, U_iou, b_iou, U_f_w, U_f_b, scale_iou, scale_c):
    n, hs = h.shape
    src = edge_index[0]
    idx_even = src[0::2]
    idx_odd = src[1::2]

    wf = U_f_w.T.astype(jnp.bfloat16)
    wi = U_iou.T.astype(jnp.bfloat16)
    wf1, wf2 = wf[:hs], wf[hs:]
    wi1, wi2 = wi[:hs], wi[hs:]
    fb = U_f_b.reshape(1, 2 * hs)
    scales = jnp.stack([scale_iou.astype(jnp.float32),
                        scale_c.astype(jnp.float32)]).reshape(1, 2)

    p = _pack_hc(h, c)

    nchunk = _NCHUNK if n % (_NCHUNK * _BN) == 0 else 1
    cn = n // nchunk
    unit = _WINDOW * _NUM_WORKERS
    cn_pad = ((cn + unit - 1) // unit) * unit
    pad = cn_pad - cn
    steps = cn_pad // _WINDOW

    out = None
    for k in range(nchunk):
        se = jax.lax.slice(idx_even, (k * cn,), ((k + 1) * cn,))
        so = jax.lax.slice(idx_odd, (k * cn,), ((k + 1) * cn,))
        se = jnp.pad(se, (0, pad)).reshape(steps, 1, _WINDOW)
        so = jnp.pad(so, (0, pad)).reshape(steps, 1, _WINDOW)
        p1, p2 = _gather_children(p, se, so, cn_pad)
        out = _tc_compute(scales, iou, p1, p2,
                          wf1, wf2, wi1, wi2, b_iou, fb,
                          rows=cn, block_offset=k * (cn // _BN),
                          prev=out, out_rows=n)
    return out


# speedup vs baseline: 2.4632x; 1.0348x over previous
import functools

import jax
import jax.numpy as jnp
from jax import lax
from jax.experimental import pallas as pl
from jax.experimental.pallas import tpu as pltpu
from jax.experimental.pallas import tpu_sc as plsc

_WINDOW = 112
_NUM_WORKERS = 32
_BN = 2000
_NCHUNK = 2
_HI = 0xFFFF0000
_RND = 0x8000


def _pack_body(h_ref, c_ref, o_ref):
    uh = lax.bitcast_convert_type(h_ref[...], jnp.uint32)
    uc = lax.bitcast_convert_type(c_ref[...], jnp.uint32)
    lo = (uh + jnp.uint32(_RND)) >> 16
    hi = (uc + jnp.uint32(_RND)) & jnp.uint32(_HI)
    o_ref[...] = lax.bitcast_convert_type(lo | hi, jnp.float32)


def _pack_hc(h, c, bn=2000):
    n, hs = h.shape
    return pl.pallas_call(
        _pack_body,
        grid=(n // bn,),
        in_specs=[pl.BlockSpec((bn, hs), lambda i: (i, 0)),
                  pl.BlockSpec((bn, hs), lambda i: (i, 0))],
        out_specs=pl.BlockSpec((bn, hs), lambda i: (i, 0)),
        out_shape=jax.ShapeDtypeStruct((n, hs), jnp.float32),
    )(h, c)


def _gather_children(p, idx_even, idx_odd, n_pad):
    n, hs = p.shape
    mesh = plsc.VectorSubcoreMesh(core_axis_name="core", subcore_axis_name="subcore")
    out_t = jax.ShapeDtypeStruct((n_pad, hs), p.dtype)

    @functools.partial(pl.kernel, out_type=(out_t, out_t), mesh=mesh,
                       scratch_types=[pltpu.SemaphoreType.DMA])
    def gather_kernel(p_hbm, ie_hbm, io_hbm, o1, o2, sem):
        def body(ie_v, io_v, o1_v, o2_v):
            a = pltpu.async_copy(p_hbm.at[ie_v.at[0, 0]], o1_v, sem)
            b = pltpu.async_copy(p_hbm.at[io_v.at[0, 0]], o2_v, sem)
            a.wait()
            b.wait()

        pltpu.emit_pipeline(
            body,
            grid=(n_pad // _WINDOW,),
            in_specs=[
                pl.BlockSpec((1, 1, _WINDOW), lambda i: (i, 0, 0)),
                pl.BlockSpec((1, 1, _WINDOW), lambda i: (i, 0, 0)),
            ],
            out_specs=[
                pl.BlockSpec((_WINDOW, hs), lambda i: (i, 0)),
                pl.BlockSpec((_WINDOW, hs), lambda i: (i, 0)),
            ],
            core_axis_name=("core", "subcore"),
            dimension_semantics=(pltpu.PARALLEL,),
        )(ie_hbm, io_hbm, o1, o2)

    return gather_kernel(p, idx_even, idx_odd)


def _tc_body(scale_ref, iou_ref, p1_ref, p2_ref,
             wf1_ref, wf2_ref, wi1_ref, wi2_ref, bi_ref, fb_ref, *rest):
    out_ref = rest[-1]
    hs = p1_ref.shape[1]
    prec = jax.lax.Precision.DEFAULT
    u1 = lax.bitcast_convert_type(p1_ref[...], jnp.uint32)
    u2 = lax.bitcast_convert_type(p2_ref[...], jnp.uint32)
    h1 = lax.bitcast_convert_type(u1 << 16, jnp.float32)
    c1 = lax.bitcast_convert_type(u1 & jnp.uint32(_HI), jnp.float32)
    h2 = lax.bitcast_convert_type(u2 << 16, jnp.float32)
    c2 = lax.bitcast_convert_type(u2 & jnp.uint32(_HI), jnp.float32)

    h1b = h1.astype(jnp.bfloat16)
    h2b = h2.astype(jnp.bfloat16)
    fpre = (jnp.dot(h1b, wf1_ref[...], preferred_element_type=jnp.float32, precision=prec)
            + jnp.dot(h2b, wf2_ref[...], preferred_element_type=jnp.float32, precision=prec)
            + fb_ref[...])
    f = jax.nn.sigmoid(fpre)
    c_red = f[:, :hs] * c1 + f[:, hs:] * c2

    mm = (jnp.dot(h1b, wi1_ref[...], preferred_element_type=jnp.float32, precision=prec)
          + jnp.dot(h2b, wi2_ref[...], preferred_element_type=jnp.float32, precision=prec))

    hnorm = jnp.sqrt(jnp.sum(h1 * h1, axis=1, keepdims=True)
                     + jnp.sum(h2 * h2, axis=1, keepdims=True))
    iou_x = iou_ref[...]
    inorm = jnp.sqrt(jnp.sum(iou_x * iou_x, axis=1, keepdims=True))
    s_iou = scale_ref[0, 0]
    s_c = scale_ref[0, 1]
    alpha = inorm * s_iou / jnp.maximum(hnorm, 1e-12)
    iou_new = mm * alpha + bi_ref[...]

    crn = jnp.sqrt(jnp.sum(c_red * c_red, axis=1, keepdims=True))
    c1n = jnp.sqrt(jnp.sum(c1 * c1, axis=1, keepdims=True))
    c_agg = c_red / jnp.maximum(crn, 1e-12) * c1n * s_c

    ig = jax.nn.sigmoid(iou_new[:, :hs])
    og = jax.nn.sigmoid(iou_new[:, hs:2 * hs])
    ug = jnp.tanh(iou_new[:, 2 * hs:])
    c_out = ig * ug + c_agg
    h_out = og * jnp.tanh(c_out)
    out_ref[:, :hs] = h_out
    out_ref[:, hs:] = c_out


def _tc_compute(scales, iou, p1, p2, wf1, wf2, wi1, wi2, bi, fb,
                rows=None, block_offset=0, prev=None, out_rows=None,
                interpret=False):
    hs = p1.shape[1]
    bn = _BN
    n = rows
    out_rows = n if out_rows is None else out_rows
    grid = (n // bn,)
    in_specs = [
        pl.BlockSpec(memory_space=pltpu.MemorySpace.SMEM),
        pl.BlockSpec((bn, 3 * hs),
                     lambda i, off=block_offset: (i + off, 0)),
        pl.BlockSpec((bn, hs), lambda i: (i, 0)),
        pl.BlockSpec((bn, hs), lambda i: (i, 0)),
        pl.BlockSpec((hs, 2 * hs), lambda i: (0, 0)),
        pl.BlockSpec((hs, 2 * hs), lambda i: (0, 0)),
        pl.BlockSpec((hs, 3 * hs), lambda i: (0, 0)),
        pl.BlockSpec((hs, 3 * hs), lambda i: (0, 0)),
        pl.BlockSpec((1, 3 * hs), lambda i: (0, 0)),
        pl.BlockSpec((1, 2 * hs), lambda i: (0, 0)),
    ]
    args = [scales, iou, p1, p2, wf1, wf2, wi1, wi2, bi, fb]
    io_aliases = {}
    if prev is not None:
        in_specs.append(pl.BlockSpec(memory_space=pl.ANY))
        args.append(prev)
        io_aliases = {10: 0}
    return pl.pallas_call(
        _tc_body,
        grid=grid,
        in_specs=in_specs,
        out_specs=pl.BlockSpec((bn, 2 * hs),
                               lambda i, off=block_offset: (i + off, 0)),
        out_shape=jax.ShapeDtypeStruct((out_rows, 2 * hs), jnp.float32),
        input_output_aliases=io_aliases,
        interpret=interpret,
    )(*args)


def kernel(iou, h, c, edge_index, U_iou, b_iou, U_f_w, U_f_b, scale_iou, scale_c):
    n, hs = h.shape
    src = edge_index[0]
    idx_even = src[0::2]
    idx_odd = src[1::2]

    wf = U_f_w.T.astype(jnp.bfloat16)
    wi = U_iou.T.astype(jnp.bfloat16)
    wf1, wf2 = wf[:hs], wf[hs:]
    wi1, wi2 = wi[:hs], wi[hs:]
    fb = U_f_b.reshape(1, 2 * hs)
    scales = jnp.stack([scale_iou.astype(jnp.float32),
                        scale_c.astype(jnp.float32)]).reshape(1, 2)

    p = _pack_hc(h, c)

    nchunk = _NCHUNK if n % (_NCHUNK * _BN) == 0 else 1
    cn = n // nchunk
    unit = _WINDOW * _NUM_WORKERS
    cn_pad = ((cn + unit - 1) // unit) * unit
    pad = cn_pad - cn
    steps = cn_pad // _WINDOW

    out = None
    for k in range(nchunk):
        se = jax.lax.slice(idx_even, (k * cn,), ((k + 1) * cn,))
        so = jax.lax.slice(idx_odd, (k * cn,), ((k + 1) * cn,))
        se = jnp.pad(se, (0, pad)).reshape(steps, 1, _WINDOW)
        so = jnp.pad(so, (0, pad)).reshape(steps, 1, _WINDOW)
        p1, p2 = _gather_children(p, se, so, cn_pad)
        out = _tc_compute(scales, iou, p1, p2,
                          wf1, wf2, wi1, wi2, b_iou, fb,
                          rows=cn, block_offset=k * (cn // _BN),
                          prev=out, out_rows=n)
    return out
